# Initial kernel scaffold; baseline (speedup 1.0000x reference)
import jax
import jax.numpy as jnp
from jax.experimental import pallas as pl


def kernel(agent_feats, edge_index, agent_pos_T, focal_indices, W1, b1, W2, b2, Wd1, bd1, Wd2, bd2):
    raise NotImplementedError("write your pallas kernel here")


# speedup vs baseline: 2.9584x; 2.9584x over previous
import numpy as np
import jax
import jax.numpy as jnp
from jax import lax
from jax.experimental import pallas as pl
from jax.experimental.pallas import tpu as pltpu, tpu_sc as plsc

NC, NS, LANES = 2, 16, 16
NW = NC * NS
DGW = 128
CH = 128
BLKC = 8

_TT = 30
_M64 = np.zeros((64, 64), np.float32)
for _t in range(_TT):
    for _s in range(_t, _TT):
        _M64[2 * _t, 2 * _s] = 1.0
        _M64[2 * _t + 1, 2 * _s + 1] = 1.0


def _mesh():
    return plsc.VectorSubcoreMesh(core_axis_name="c", subcore_axis_name="s",
                                  num_cores=NC, num_subcores=NS)


def _sc_aggregate(NP, D, EP):
    EPW = EP // NW
    NCH = EPW // CH
    NBLK = NCH // BLKC
    NPAIR = NBLK // 2
    RPT = NP // NS

    out_type = jax.ShapeDtypeStruct((NC, NP, D), jnp.float32)
    scratch = [
        pltpu.VMEM((BLKC, CH), jnp.int32),
        pltpu.VMEM((BLKC, CH), jnp.int32),
        pltpu.VMEM((BLKC, CH), jnp.int32),
        pltpu.VMEM((BLKC, CH), jnp.int32),
        pltpu.VMEM((CH, D), jnp.float32),
        pltpu.VMEM((CH, D), jnp.float32),
        pltpu.VMEM_SHARED((NP, D), jnp.float32),
        pltpu.SemaphoreType.DMA,
        pltpu.SemaphoreType.DMA,
        pltpu.SemaphoreType.DMA,
        pltpu.SemaphoreType.DMA,
    ]

    def body(x_hbm, ei_hbm, z_hbm, acc_out,
             sb0, sb1, db0, db1, rows0, rows1, acc_sh,
             isem0, isem1, gsem0, gsem1):
        c = lax.axis_index("c")
        s = lax.axis_index("s")
        wid = c * NS + s
        rows = (rows0, rows1)
        gsem = (gsem0, gsem1)
        srcb = (sb0, sb1)
        dstb = (db0, db1)
        isem = (isem0, isem1)

        pltpu.sync_copy(z_hbm, acc_sh.at[pl.ds(s * RPT, RPT), :])

        def idx_start(b, par):
            pltpu.async_copy(ei_hbm.at[0, wid, pl.ds(b * BLKC, BLKC)],
                             srcb[par], isem[par])
            pltpu.async_copy(ei_hbm.at[1, wid, pl.ds(b * BLKC, BLKC)],
                             dstb[par], isem[par])

        def idx_wait(b, par):
            pltpu.make_async_copy(ei_hbm.at[0, wid, pl.ds(b * BLKC, BLKC)],
                                  srcb[par], isem[par]).wait()
            pltpu.make_async_copy(ei_hbm.at[1, wid, pl.ds(b * BLKC, BLKC)],
                                  dstb[par], isem[par]).wait()

        def g_start(par, k, rp):
            pltpu.async_copy(x_hbm.at[srcb[par].at[k]], rows[rp], gsem[rp])

        def g_wait(par, k, rp):
            pltpu.make_async_copy(x_hbm.at[srcb[par].at[k]], rows[rp],
                                  gsem[rp]).wait()

        idx_start(0, 0)
        idx_start(1, 1)
        plsc.subcore_barrier()

        def pair(p, _):
            b0 = 2 * p
            b1 = 2 * p + 1

            @pl.when(p == 0)
            def _():
                idx_wait(b0, 0)
                g_start(0, 0, 0)

            for par in (0, 1):
                b = b0 if par == 0 else b1
                for k in range(BLKC):
                    rp = k & 1
                    g_wait(par, k, rp)
                    if k < BLKC - 1:
                        g_start(par, k + 1, 1 - rp)
                    elif par == 0:
                        idx_wait(b1, 1)
                        g_start(1, 0, 1 - rp)
                    else:
                        @pl.when(p < NPAIR - 1)
                        def _():
                            idx_wait(b1 + 1, 0)
                            g_start(0, 0, 1 - rp)
                    pltpu.sync_copy(rows[rp], acc_sh.at[dstb[par].at[k]],
                                    add=True)
                @pl.when(p < NPAIR - 1)
                def _():
                    idx_start(b + 2, par)
            return 0
        lax.fori_loop(0, NPAIR, pair, 0)

        plsc.subcore_barrier()
        pltpu.sync_copy(acc_sh.at[pl.ds(s * RPT, RPT), :],
                        acc_out.at[c, pl.ds(s * RPT, RPT), :])

    return pl.kernel(body, out_type=out_type, mesh=_mesh(),
                     scratch_types=tuple(scratch))


def _sc_degree(NP, EP):
    EPW = EP // NW
    NCH = EPW // CH
    NBLK = NCH // BLKC
    RPT = NP // NS

    out_type = jax.ShapeDtypeStruct((NC, NP, DGW), jnp.float32)
    scratch = [
        pltpu.VMEM((BLKC, CH), jnp.int32),
        pltpu.VMEM((BLKC, CH), jnp.int32),
        pltpu.VMEM((CH, DGW), jnp.float32),
        pltpu.VMEM_SHARED((NP, DGW), jnp.float32),
        pltpu.SemaphoreType.DMA,
        pltpu.SemaphoreType.DMA,
    ]

    def body(ei_hbm, zd_hbm, ones_hbm, deg_out,
             db0, db1, ones_v, deg_sh, isem0, isem1):
        c = lax.axis_index("c")
        s = lax.axis_index("s")
        wid = c * NS + s
        dstb = (db0, db1)
        isem = (isem0, isem1)

        pltpu.sync_copy(ones_hbm, ones_v)
        pltpu.sync_copy(zd_hbm, deg_sh.at[pl.ds(s * RPT, RPT), :])

        def idx_start(b, par):
            pltpu.async_copy(ei_hbm.at[1, wid, pl.ds(b * BLKC, BLKC)],
                             dstb[par], isem[par])

        def idx_wait(b, par):
            pltpu.make_async_copy(ei_hbm.at[1, wid, pl.ds(b * BLKC, BLKC)],
                                  dstb[par], isem[par]).wait()

        idx_start(0, 0)
        idx_start(1, 1)
        plsc.subcore_barrier()

        def pair(p, _):
            for par in (0, 1):
                b = 2 * p + par
                idx_wait(b, par)
                for k in range(BLKC):
                    pltpu.sync_copy(ones_v, deg_sh.at[dstb[par].at[k]],
                                    add=True)

                @pl.when(p < NBLK // 2 - 1)
                def _():
                    idx_start(b + 2, par)
            return 0
        lax.fori_loop(0, NBLK // 2, pair, 0)

        plsc.subcore_barrier()
        pltpu.sync_copy(deg_sh.at[pl.ds(s * RPT, RPT), :],
                        deg_out.at[c, pl.ds(s * RPT, RPT), :])

    return pl.kernel(body, out_type=out_type, mesh=_mesh(),
                     scratch_types=tuple(scratch))


def _tc_layer(NP, D, BLK):
    NB = NP // BLK

    def body(acc_ref, deg_ref, x_ref, w_ref, b_ref, o_ref):
        a = acc_ref[0] + acc_ref[1]
        d = deg_ref[0, :, 0] + deg_ref[1, :, 0]
        d = jnp.maximum(d, 1.0)
        z = a / d[:, None] + x_ref[...]
        z = jnp.dot(z, w_ref[...], preferred_element_type=jnp.float32)
        o_ref[...] = jnp.maximum(z + b_ref[...], 0.0)

    return pl.pallas_call(
        body,
        grid=(NB,),
        in_specs=[
            pl.BlockSpec((NC, BLK, D), lambda i: (0, i, 0)),
            pl.BlockSpec((NC, BLK, DGW), lambda i: (0, i, 0)),
            pl.BlockSpec((BLK, D), lambda i: (i, 0)),
            pl.BlockSpec((D, D), lambda i: (0, 0)),
            pl.BlockSpec((1, D), lambda i: (0, 0)),
        ],
        out_specs=pl.BlockSpec((BLK, D), lambda i: (i, 0)),
        out_shape=jax.ShapeDtypeStruct((NP, D), jnp.float32),
    )


def _sc_gather_focal(NP, D, F):
    RW = F // NS
    out_type = (jax.ShapeDtypeStruct((F, D), jnp.float32),
                jax.ShapeDtypeStruct((F, D), jnp.float32))
    scratch = (
        pltpu.VMEM((RW,), jnp.int32),
        pltpu.VMEM((RW, D), jnp.float32),
        pltpu.VMEM((RW, D), jnp.float32),
        pltpu.SemaphoreType.DMA,
    )

    def body(am_hbm, pos_hbm, focal_hbm, ff_out, sp_out,
             idx_v, rbuf, pbuf, sem):
        c = lax.axis_index("c")
        s = lax.axis_index("s")
        wid = c * NS + s

        @pl.when(wid < NS)
        def _():
            pltpu.sync_copy(focal_hbm.at[pl.ds(wid * RW, RW)], idx_v)
            pltpu.async_copy(am_hbm.at[idx_v], rbuf, sem).wait()
            pltpu.sync_copy(rbuf, ff_out.at[pl.ds(wid * RW, RW), :])

        @pl.when(wid >= NS)
        def _():
            w = wid - NS
            pltpu.sync_copy(focal_hbm.at[pl.ds(w * RW, RW)], idx_v)
            pltpu.async_copy(pos_hbm.at[idx_v], pbuf, sem).wait()
            pltpu.sync_copy(pbuf, sp_out.at[pl.ds(w * RW, RW), :])

    return pl.kernel(body, out_type=out_type, mesh=_mesh(),
                     scratch_types=scratch)


def _tc_decoder(F):
    def body(ff, st, w1, b1, w2, b2, m, o_ref):
        h1 = jnp.dot(ff[...], w1[...], preferred_element_type=jnp.float32)
        h1 = jnp.maximum(h1 + b1[...], 0.0)
        dp = jnp.dot(h1, w2[...], preferred_element_type=jnp.float32) + b2[...]
        o_ref[...] = st[...] + jnp.dot(dp, m[...],
                                       preferred_element_type=jnp.float32)

    return pl.pallas_call(
        body, out_shape=jax.ShapeDtypeStruct((F, 64), jnp.float32))


def kernel(agent_feats, edge_index, agent_pos_T, focal_indices,
           W1, b1, W2, b2, Wd1, bd1, Wd2, bd2):
    N, D = agent_feats.shape
    E = edge_index.shape[1]
    F = focal_indices.shape[0]
    T = Wd2.shape[1] // 2

    NP = -(-N // (NS * CH)) * NS * CH
    EGRAN = NW * CH * BLKC * 2
    EP = -(-E // EGRAN) * EGRAN

    x = jnp.pad(agent_feats, ((0, NP - N), (0, 0)))
    fill = jnp.full((2, EP - E), NP - 1, jnp.int32)
    ei = jnp.concatenate([edge_index.astype(jnp.int32), fill], axis=1)
    ei = ei.reshape(2, NW, EP // (NW * CH * BLKC), BLKC, CH)
    ei = ei.reshape(2, NW, EP // (NW * CH), CH)
    pos = jnp.tile(jnp.pad(agent_pos_T, ((0, NP - N), (0, 0))), (1, D // 2))

    RPT = NP // NS
    zz = jnp.zeros((RPT, D), jnp.float32)
    zd = jnp.zeros((RPT, DGW), jnp.float32)
    on1 = jnp.ones((CH, DGW), jnp.float32)

    deg = _sc_degree(NP, EP)(ei, zd, on1)
    acc1 = _sc_aggregate(NP, D, EP)(x, ei, zz)
    layer = _tc_layer(NP, D, 1024)
    h = layer(acc1, deg, x, W1, b1.reshape(1, D))
    acc2 = _sc_aggregate(NP, D, EP)(h, ei, zz)
    amap = layer(acc2, deg, h, W2, b2.reshape(1, D))

    ff, sp = _sc_gather_focal(NP, D, F)(amap, pos, focal_indices)

    st64 = sp[:, :64]
    w2p = jnp.pad(Wd2, ((0, 0), (0, 4)))
    b2p = jnp.pad(bd2, (0, 4)).reshape(1, 64)
    out = _tc_decoder(F)(ff, st64, Wd1, bd1.reshape(1, Wd1.shape[1]),
                         w2p, b2p, jnp.asarray(_M64))
    pred_traj = out[:, :2 * T].reshape(F, T, 2)
    return (pred_traj, focal_indices)


# speedup vs baseline: 3.0383x; 1.0270x over previous
import numpy as np
import jax
import jax.numpy as jnp
from jax import lax
from jax.experimental import pallas as pl
from jax.experimental.pallas import tpu as pltpu, tpu_sc as plsc

NC, NS, LANES = 2, 16, 16
NW = NC * NS
DGW = 128
CH = 64
BLKC = 16
RING = 4

_TT = 30
_M64 = np.zeros((64, 64), np.float32)
for _t in range(_TT):
    for _s in range(_t, _TT):
        _M64[2 * _t, 2 * _s] = 1.0
        _M64[2 * _t + 1, 2 * _s + 1] = 1.0


def _mesh():
    return plsc.VectorSubcoreMesh(core_axis_name="c", subcore_axis_name="s",
                                  num_cores=NC, num_subcores=NS)


def _sc_aggregate(NP, D, EP):
    EPW = EP // NW
    NCH = EPW // CH
    NBLK = NCH // BLKC
    NPAIR = NBLK // 2
    RPT = NP // NS

    LA = RING - 1
    out_type = jax.ShapeDtypeStruct((NC, NP, D), jnp.float32)
    scratch = (
        [pltpu.VMEM((BLKC, CH), jnp.int32) for _ in range(2)]
        + [pltpu.VMEM((BLKC, CH), jnp.int32) for _ in range(2)]
        + [pltpu.VMEM((CH, D), jnp.float32) for _ in range(RING)]
        + [pltpu.VMEM_SHARED((NP, D), jnp.float32)]
        + [pltpu.SemaphoreType.DMA for _ in range(2 + RING)]
    )

    def body(x_hbm, ei_hbm, z_hbm, acc_out, *rest):
        srcb = rest[0:2]
        dstb = rest[2:4]
        rows = rest[4:4 + RING]
        acc_sh = rest[4 + RING]
        isem = rest[5 + RING:7 + RING]
        gsem = rest[7 + RING:]
        c = lax.axis_index("c")
        s = lax.axis_index("s")
        wid = c * NS + s

        pltpu.sync_copy(z_hbm, acc_sh.at[pl.ds(s * RPT, RPT), :])

        def idx_start(b, par):
            pltpu.async_copy(ei_hbm.at[0, wid, pl.ds(b * BLKC, BLKC)],
                             srcb[par], isem[par])
            pltpu.async_copy(ei_hbm.at[1, wid, pl.ds(b * BLKC, BLKC)],
                             dstb[par], isem[par])

        def idx_wait(b, par):
            pltpu.make_async_copy(ei_hbm.at[0, wid, pl.ds(b * BLKC, BLKC)],
                                  srcb[par], isem[par]).wait()
            pltpu.make_async_copy(ei_hbm.at[1, wid, pl.ds(b * BLKC, BLKC)],
                                  dstb[par], isem[par]).wait()

        def g_start(par, k, rp):
            pltpu.async_copy(x_hbm.at[srcb[par].at[k]], rows[rp], gsem[rp])

        def g_wait(par, k, rp):
            pltpu.make_async_copy(x_hbm.at[srcb[par].at[k]], rows[rp],
                                  gsem[rp]).wait()

        idx_start(0, 0)
        idx_start(1, 1)
        plsc.subcore_barrier()
        idx_wait(0, 0)
        for k in range(LA):
            g_start(0, k, k & (RING - 1))

        def pair(p, _):
            for par in (0, 1):
                b = 2 * p + par
                for k in range(BLKC):
                    rp = k & (RING - 1)
                    g_wait(par, k, rp)
                    t = k + LA
                    tp = t & (RING - 1)
                    if t < BLKC:
                        g_start(par, t, tp)
                    elif par == 0:
                        if t == BLKC:
                            idx_wait(b + 1, 1)
                        g_start(1, t - BLKC, tp)
                    else:
                        @pl.when(p < NPAIR - 1)
                        def _():
                            if t == BLKC:
                                idx_wait(b + 1, 0)
                            g_start(0, t - BLKC, tp)
                    pltpu.sync_copy(rows[rp], acc_sh.at[dstb[par].at[k]],
                                    add=True)
                @pl.when(p < NPAIR - 1)
                def _():
                    idx_start(b + 2, par)
            return 0
        lax.fori_loop(0, NPAIR, pair, 0)

        plsc.subcore_barrier()
        pltpu.sync_copy(acc_sh.at[pl.ds(s * RPT, RPT), :],
                        acc_out.at[c, pl.ds(s * RPT, RPT), :])

    return pl.kernel(body, out_type=out_type, mesh=_mesh(),
                     scratch_types=tuple(scratch))


def _sc_degree(NP, EP):
    EPW = EP // NW
    NCH = EPW // CH
    NBLK = NCH // BLKC
    RPT = NP // NS

    out_type = jax.ShapeDtypeStruct((NC, NP, DGW), jnp.float32)
    scratch = [
        pltpu.VMEM((BLKC, CH), jnp.int32),
        pltpu.VMEM((BLKC, CH), jnp.int32),
        pltpu.VMEM((CH, DGW), jnp.float32),
        pltpu.VMEM_SHARED((NP, DGW), jnp.float32),
        pltpu.SemaphoreType.DMA,
        pltpu.SemaphoreType.DMA,
    ]

    def body(ei_hbm, zd_hbm, ones_hbm, deg_out,
             db0, db1, ones_v, deg_sh, isem0, isem1):
        c = lax.axis_index("c")
        s = lax.axis_index("s")
        wid = c * NS + s
        dstb = (db0, db1)
        isem = (isem0, isem1)

        pltpu.sync_copy(ones_hbm, ones_v)
        pltpu.sync_copy(zd_hbm, deg_sh.at[pl.ds(s * RPT, RPT), :])

        def idx_start(b, par):
            pltpu.async_copy(ei_hbm.at[1, wid, pl.ds(b * BLKC, BLKC)],
                             dstb[par], isem[par])

        def idx_wait(b, par):
            pltpu.make_async_copy(ei_hbm.at[1, wid, pl.ds(b * BLKC, BLKC)],
                                  dstb[par], isem[par]).wait()

        idx_start(0, 0)
        idx_start(1, 1)
        plsc.subcore_barrier()

        def pair(p, _):
            for par in (0, 1):
                b = 2 * p + par
                idx_wait(b, par)
                for k in range(BLKC):
                    pltpu.sync_copy(ones_v, deg_sh.at[dstb[par].at[k]],
                                    add=True)

                @pl.when(p < NBLK // 2 - 1)
                def _():
                    idx_start(b + 2, par)
            return 0
        lax.fori_loop(0, NBLK // 2, pair, 0)

        plsc.subcore_barrier()
        pltpu.sync_copy(deg_sh.at[pl.ds(s * RPT, RPT), :],
                        deg_out.at[c, pl.ds(s * RPT, RPT), :])

    return pl.kernel(body, out_type=out_type, mesh=_mesh(),
                     scratch_types=tuple(scratch))


def _tc_layer(NP, D, BLK):
    NB = NP // BLK

    def body(acc_ref, deg_ref, x_ref, w_ref, b_ref, o_ref):
        a = acc_ref[0] + acc_ref[1]
        d = deg_ref[0, :, 0] + deg_ref[1, :, 0]
        d = jnp.maximum(d, 1.0)
        z = a / d[:, None] + x_ref[...]
        z = jnp.dot(z, w_ref[...], preferred_element_type=jnp.float32)
        o_ref[...] = jnp.maximum(z + b_ref[...], 0.0)

    return pl.pallas_call(
        body,
        grid=(NB,),
        in_specs=[
            pl.BlockSpec((NC, BLK, D), lambda i: (0, i, 0)),
            pl.BlockSpec((NC, BLK, DGW), lambda i: (0, i, 0)),
            pl.BlockSpec((BLK, D), lambda i: (i, 0)),
            pl.BlockSpec((D, D), lambda i: (0, 0)),
            pl.BlockSpec((1, D), lambda i: (0, 0)),
        ],
        out_specs=pl.BlockSpec((BLK, D), lambda i: (i, 0)),
        out_shape=jax.ShapeDtypeStruct((NP, D), jnp.float32),
    )


def _sc_gather_focal(NP, D, F):
    RW = F // NS
    out_type = (jax.ShapeDtypeStruct((F, D), jnp.float32),
                jax.ShapeDtypeStruct((F, D), jnp.float32))
    scratch = (
        pltpu.VMEM((RW,), jnp.int32),
        pltpu.VMEM((RW, D), jnp.float32),
        pltpu.VMEM((RW, D), jnp.float32),
        pltpu.SemaphoreType.DMA,
    )

    def body(am_hbm, pos_hbm, focal_hbm, ff_out, sp_out,
             idx_v, rbuf, pbuf, sem):
        c = lax.axis_index("c")
        s = lax.axis_index("s")
        wid = c * NS + s

        @pl.when(wid < NS)
        def _():
            pltpu.sync_copy(focal_hbm.at[pl.ds(wid * RW, RW)], idx_v)
            pltpu.async_copy(am_hbm.at[idx_v], rbuf, sem).wait()
            pltpu.sync_copy(rbuf, ff_out.at[pl.ds(wid * RW, RW), :])

        @pl.when(wid >= NS)
        def _():
            w = wid - NS
            pltpu.sync_copy(focal_hbm.at[pl.ds(w * RW, RW)], idx_v)
            pltpu.async_copy(pos_hbm.at[idx_v], pbuf, sem).wait()
            pltpu.sync_copy(pbuf, sp_out.at[pl.ds(w * RW, RW), :])

    return pl.kernel(body, out_type=out_type, mesh=_mesh(),
                     scratch_types=scratch)


def _tc_decoder(F):
    def body(ff, st, w1, b1, w2, b2, m, o_ref):
        h1 = jnp.dot(ff[...], w1[...], preferred_element_type=jnp.float32)
        h1 = jnp.maximum(h1 + b1[...], 0.0)
        dp = jnp.dot(h1, w2[...], preferred_element_type=jnp.float32) + b2[...]
        o_ref[...] = st[...] + jnp.dot(dp, m[...],
                                       preferred_element_type=jnp.float32)

    return pl.pallas_call(
        body, out_shape=jax.ShapeDtypeStruct((F, 64), jnp.float32))


def kernel(agent_feats, edge_index, agent_pos_T, focal_indices,
           W1, b1, W2, b2, Wd1, bd1, Wd2, bd2):
    N, D = agent_feats.shape
    E = edge_index.shape[1]
    F = focal_indices.shape[0]
    T = Wd2.shape[1] // 2

    NP = -(-N // (NS * CH)) * NS * CH
    EGRAN = NW * CH * BLKC * 2
    EP = -(-E // EGRAN) * EGRAN

    x = jnp.pad(agent_feats, ((0, NP - N), (0, 0)))
    fill = jnp.full((2, EP - E), NP - 1, jnp.int32)
    ei = jnp.concatenate([edge_index.astype(jnp.int32), fill], axis=1)
    ei = ei.reshape(2, NW, EP // (NW * CH * BLKC), BLKC, CH)
    ei = ei.reshape(2, NW, EP // (NW * CH), CH)
    pos = jnp.tile(jnp.pad(agent_pos_T, ((0, NP - N), (0, 0))), (1, D // 2))

    RPT = NP // NS
    zz = jnp.zeros((RPT, D), jnp.float32)
    zd = jnp.zeros((RPT, DGW), jnp.float32)
    on1 = jnp.ones((CH, DGW), jnp.float32)

    deg = _sc_degree(NP, EP)(ei, zd, on1)
    acc1 = _sc_aggregate(NP, D, EP)(x, ei, zz)
    layer = _tc_layer(NP, D, 1024)
    h = layer(acc1, deg, x, W1, b1.reshape(1, D))
    acc2 = _sc_aggregate(NP, D, EP)(h, ei, zz)
    amap = layer(acc2, deg, h, W2, b2.reshape(1, D))

    ff, sp = _sc_gather_focal(NP, D, F)(amap, pos, focal_indices)

    st64 = sp[:, :64]
    w2p = jnp.pad(Wd2, ((0, 0), (0, 4)))
    b2p = jnp.pad(bd2, (0, 4)).reshape(1, 64)
    out = _tc_decoder(F)(ff, st64, Wd1, bd1.reshape(1, Wd1.shape[1]),
                         w2p, b2p, jnp.asarray(_M64))
    pred_traj = out[:, :2 * T].reshape(F, T, 2)
    return (pred_traj, focal_indices)


# speedup vs baseline: 3.0897x; 1.0169x over previous
import numpy as np
import jax
import jax.numpy as jnp
from jax import lax
from jax.experimental import pallas as pl
from jax.experimental.pallas import tpu as pltpu, tpu_sc as plsc

NC, NS, LANES = 2, 16, 16
NW = NC * NS
DGW = 128
CH = 32
BLKC = 16
RING = 8

_TT = 30
_M64 = np.zeros((64, 64), np.float32)
for _t in range(_TT):
    for _s in range(_t, _TT):
        _M64[2 * _t, 2 * _s] = 1.0
        _M64[2 * _t + 1, 2 * _s + 1] = 1.0


def _mesh():
    return plsc.VectorSubcoreMesh(core_axis_name="c", subcore_axis_name="s",
                                  num_cores=NC, num_subcores=NS)


def _sc_aggregate(NP, D, EP):
    EPW = EP // NW
    NCH = EPW // CH
    NBLK = NCH // BLKC
    NPAIR = NBLK // 2
    RPT = NP // NS

    LA = RING - 1
    out_type = jax.ShapeDtypeStruct((NC, NP, D), jnp.float32)
    scratch = (
        [pltpu.VMEM((BLKC, CH), jnp.int32) for _ in range(2)]
        + [pltpu.VMEM((BLKC, CH), jnp.int32) for _ in range(2)]
        + [pltpu.VMEM((CH, D), jnp.float32) for _ in range(RING)]
        + [pltpu.VMEM_SHARED((NP, D), jnp.float32)]
        + [pltpu.SemaphoreType.DMA for _ in range(2 + RING)]
    )

    def body(x_hbm, ei_hbm, z_hbm, acc_out, *rest):
        srcb = rest[0:2]
        dstb = rest[2:4]
        rows = rest[4:4 + RING]
        acc_sh = rest[4 + RING]
        isem = rest[5 + RING:7 + RING]
        gsem = rest[7 + RING:]
        c = lax.axis_index("c")
        s = lax.axis_index("s")
        wid = c * NS + s

        pltpu.sync_copy(z_hbm, acc_sh.at[pl.ds(s * RPT, RPT), :])

        def idx_start(b, par):
            pltpu.async_copy(ei_hbm.at[0, wid, pl.ds(b * BLKC, BLKC)],
                             srcb[par], isem[par])
            pltpu.async_copy(ei_hbm.at[1, wid, pl.ds(b * BLKC, BLKC)],
                             dstb[par], isem[par])

        def idx_wait(b, par):
            pltpu.make_async_copy(ei_hbm.at[0, wid, pl.ds(b * BLKC, BLKC)],
                                  srcb[par], isem[par]).wait()
            pltpu.make_async_copy(ei_hbm.at[1, wid, pl.ds(b * BLKC, BLKC)],
                                  dstb[par], isem[par]).wait()

        def g_start(par, k, rp):
            pltpu.async_copy(x_hbm.at[srcb[par].at[k]], rows[rp], gsem[rp])

        def g_wait(par, k, rp):
            pltpu.make_async_copy(x_hbm.at[srcb[par].at[k]], rows[rp],
                                  gsem[rp]).wait()

        idx_start(0, 0)
        idx_start(1, 1)
        plsc.subcore_barrier()
        idx_wait(0, 0)
        for k in range(LA):
            g_start(0, k, k & (RING - 1))

        def pair(p, _):
            for par in (0, 1):
                b = 2 * p + par
                for k in range(BLKC):
                    rp = k & (RING - 1)
                    g_wait(par, k, rp)
                    t = k + LA
                    tp = t & (RING - 1)
                    if t < BLKC:
                        g_start(par, t, tp)
                    elif par == 0:
                        if t == BLKC:
                            idx_wait(b + 1, 1)
                        g_start(1, t - BLKC, tp)
                    else:
                        @pl.when(p < NPAIR - 1)
                        def _():
                            if t == BLKC:
                                idx_wait(b + 1, 0)
                            g_start(0, t - BLKC, tp)
                    pltpu.sync_copy(rows[rp], acc_sh.at[dstb[par].at[k]],
                                    add=True)
                @pl.when(p < NPAIR - 1)
                def _():
                    idx_start(b + 2, par)
            return 0
        lax.fori_loop(0, NPAIR, pair, 0)

        plsc.subcore_barrier()
        pltpu.sync_copy(acc_sh.at[pl.ds(s * RPT, RPT), :],
                        acc_out.at[c, pl.ds(s * RPT, RPT), :])

    return pl.kernel(body, out_type=out_type, mesh=_mesh(),
                     scratch_types=tuple(scratch))


def _sc_degree(NP, EP):
    EPW = EP // NW
    NCH = EPW // CH
    NBLK = NCH // BLKC
    RPT = NP // NS

    out_type = jax.ShapeDtypeStruct((NC, NP, DGW), jnp.float32)
    scratch = [
        pltpu.VMEM((BLKC, CH), jnp.int32),
        pltpu.VMEM((BLKC, CH), jnp.int32),
        pltpu.VMEM((CH, DGW), jnp.float32),
        pltpu.VMEM_SHARED((NP, DGW), jnp.float32),
        pltpu.SemaphoreType.DMA,
        pltpu.SemaphoreType.DMA,
    ]

    def body(ei_hbm, zd_hbm, ones_hbm, deg_out,
             db0, db1, ones_v, deg_sh, isem0, isem1):
        c = lax.axis_index("c")
        s = lax.axis_index("s")
        wid = c * NS + s
        dstb = (db0, db1)
        isem = (isem0, isem1)

        pltpu.sync_copy(ones_hbm, ones_v)
        pltpu.sync_copy(zd_hbm, deg_sh.at[pl.ds(s * RPT, RPT), :])

        def idx_start(b, par):
            pltpu.async_copy(ei_hbm.at[1, wid, pl.ds(b * BLKC, BLKC)],
                             dstb[par], isem[par])

        def idx_wait(b, par):
            pltpu.make_async_copy(ei_hbm.at[1, wid, pl.ds(b * BLKC, BLKC)],
                                  dstb[par], isem[par]).wait()

        idx_start(0, 0)
        idx_start(1, 1)
        plsc.subcore_barrier()

        def pair(p, _):
            for par in (0, 1):
                b = 2 * p + par
                idx_wait(b, par)
                for k in range(BLKC):
                    pltpu.sync_copy(ones_v, deg_sh.at[dstb[par].at[k]],
                                    add=True)

                @pl.when(p < NBLK // 2 - 1)
                def _():
                    idx_start(b + 2, par)
            return 0
        lax.fori_loop(0, NBLK // 2, pair, 0)

        plsc.subcore_barrier()
        pltpu.sync_copy(deg_sh.at[pl.ds(s * RPT, RPT), :],
                        deg_out.at[c, pl.ds(s * RPT, RPT), :])

    return pl.kernel(body, out_type=out_type, mesh=_mesh(),
                     scratch_types=tuple(scratch))


def _tc_layer(NP, D, BLK):
    NB = NP // BLK

    def body(acc_ref, deg_ref, x_ref, w_ref, b_ref, o_ref):
        a = acc_ref[0] + acc_ref[1]
        d = deg_ref[0, :, 0] + deg_ref[1, :, 0]
        d = jnp.maximum(d, 1.0)
        z = a / d[:, None] + x_ref[...]
        z = jnp.dot(z, w_ref[...], preferred_element_type=jnp.float32)
        o_ref[...] = jnp.maximum(z + b_ref[...], 0.0)

    return pl.pallas_call(
        body,
        grid=(NB,),
        in_specs=[
            pl.BlockSpec((NC, BLK, D), lambda i: (0, i, 0)),
            pl.BlockSpec((NC, BLK, DGW), lambda i: (0, i, 0)),
            pl.BlockSpec((BLK, D), lambda i: (i, 0)),
            pl.BlockSpec((D, D), lambda i: (0, 0)),
            pl.BlockSpec((1, D), lambda i: (0, 0)),
        ],
        out_specs=pl.BlockSpec((BLK, D), lambda i: (i, 0)),
        out_shape=jax.ShapeDtypeStruct((NP, D), jnp.float32),
    )


def _sc_gather_focal(NP, D, F):
    RW = F // NS
    out_type = (jax.ShapeDtypeStruct((F, D), jnp.float32),
                jax.ShapeDtypeStruct((F, D), jnp.float32))
    scratch = (
        pltpu.VMEM((RW,), jnp.int32),
        pltpu.VMEM((RW, D), jnp.float32),
        pltpu.VMEM((RW, D), jnp.float32),
        pltpu.SemaphoreType.DMA,
    )

    def body(am_hbm, pos_hbm, focal_hbm, ff_out, sp_out,
             idx_v, rbuf, pbuf, sem):
        c = lax.axis_index("c")
        s = lax.axis_index("s")
        wid = c * NS + s

        @pl.when(wid < NS)
        def _():
            pltpu.sync_copy(focal_hbm.at[pl.ds(wid * RW, RW)], idx_v)
            pltpu.async_copy(am_hbm.at[idx_v], rbuf, sem).wait()
            pltpu.sync_copy(rbuf, ff_out.at[pl.ds(wid * RW, RW), :])

        @pl.when(wid >= NS)
        def _():
            w = wid - NS
            pltpu.sync_copy(focal_hbm.at[pl.ds(w * RW, RW)], idx_v)
            pltpu.async_copy(pos_hbm.at[idx_v], pbuf, sem).wait()
            pltpu.sync_copy(pbuf, sp_out.at[pl.ds(w * RW, RW), :])

    return pl.kernel(body, out_type=out_type, mesh=_mesh(),
                     scratch_types=scratch)


def _tc_decoder(F):
    def body(ff, st, w1, b1, w2, b2, m, o_ref):
        h1 = jnp.dot(ff[...], w1[...], preferred_element_type=jnp.float32)
        h1 = jnp.maximum(h1 + b1[...], 0.0)
        dp = jnp.dot(h1, w2[...], preferred_element_type=jnp.float32) + b2[...]
        o_ref[...] = st[...] + jnp.dot(dp, m[...],
                                       preferred_element_type=jnp.float32)

    return pl.pallas_call(
        body, out_shape=jax.ShapeDtypeStruct((F, 64), jnp.float32))


def kernel(agent_feats, edge_index, agent_pos_T, focal_indices,
           W1, b1, W2, b2, Wd1, bd1, Wd2, bd2):
    N, D = agent_feats.shape
    E = edge_index.shape[1]
    F = focal_indices.shape[0]
    T = Wd2.shape[1] // 2

    NP = -(-N // (NS * CH)) * NS * CH
    EGRAN = NW * CH * BLKC * 2
    EP = -(-E // EGRAN) * EGRAN

    x = jnp.pad(agent_feats, ((0, NP - N), (0, 0)))
    fill = jnp.full((2, EP - E), NP - 1, jnp.int32)
    ei = jnp.concatenate([edge_index.astype(jnp.int32), fill], axis=1)
    ei = ei.reshape(2, NW, EP // (NW * CH * BLKC), BLKC, CH)
    ei = ei.reshape(2, NW, EP // (NW * CH), CH)
    pos = jnp.tile(jnp.pad(agent_pos_T, ((0, NP - N), (0, 0))), (1, D // 2))

    RPT = NP // NS
    zz = jnp.zeros((RPT, D), jnp.float32)
    zd = jnp.zeros((RPT, DGW), jnp.float32)
    on1 = jnp.ones((CH, DGW), jnp.float32)

    deg = _sc_degree(NP, EP)(ei, zd, on1)
    acc1 = _sc_aggregate(NP, D, EP)(x, ei, zz)
    layer = _tc_layer(NP, D, 1024)
    h = layer(acc1, deg, x, W1, b1.reshape(1, D))
    acc2 = _sc_aggregate(NP, D, EP)(h, ei, zz)
    amap = layer(acc2, deg, h, W2, b2.reshape(1, D))

    ff, sp = _sc_gather_focal(NP, D, F)(amap, pos, focal_indices)

    st64 = sp[:, :64]
    w2p = jnp.pad(Wd2, ((0, 0), (0, 4)))
    b2p = jnp.pad(bd2, (0, 4)).reshape(1, 64)
    out = _tc_decoder(F)(ff, st64, Wd1, bd1.reshape(1, Wd1.shape[1]),
                         w2p, b2p, jnp.asarray(_M64))
    pred_traj = out[:, :2 * T].reshape(F, T, 2)
    return (pred_traj, focal_indices)


# speedup vs baseline: 3.1250x; 1.0114x over previous
import numpy as np
import jax
import jax.numpy as jnp
from jax import lax
from jax.experimental import pallas as pl
from jax.experimental.pallas import tpu as pltpu, tpu_sc as plsc

NC, NS, LANES = 2, 16, 16
NW = NC * NS
DGW = 128
CH = 32
BLKC = 16
RING = 8
PAIR0_FRAC = 0.2

_TT = 30
_M64 = np.zeros((64, 64), np.float32)
for _t in range(_TT):
    for _s in range(_t, _TT):
        _M64[2 * _t, 2 * _s] = 1.0
        _M64[2 * _t + 1, 2 * _s + 1] = 1.0


def _mesh():
    return plsc.VectorSubcoreMesh(core_axis_name="c", subcore_axis_name="s",
                                  num_cores=NC, num_subcores=NS)


def _sc_aggregate(NP, D, EP, pair0_frac):
    R = EP // CH
    GRAIN = NS * BLKC * 2
    UNITS = R // GRAIN
    NPAIR0 = max(1, min(UNITS - 1, round(UNITS * pair0_frac)))
    NPAIR1 = UNITS - NPAIR0
    R0 = NPAIR0 * GRAIN
    RPT = NP // NS

    LA = RING - 1
    out_type = jax.ShapeDtypeStruct((NC, NP, D), jnp.float32)
    scratch = (
        [pltpu.VMEM((BLKC, CH), jnp.int32) for _ in range(2)]
        + [pltpu.VMEM((BLKC, CH), jnp.int32) for _ in range(2)]
        + [pltpu.VMEM((CH, D), jnp.float32) for _ in range(RING)]
        + [pltpu.VMEM_SHARED((NP, D), jnp.float32)]
        + [pltpu.SemaphoreType.DMA for _ in range(2 + RING)]
    )

    def body(x_hbm, es_hbm, ed_hbm, z_hbm, acc_out, *rest):
        srcb = rest[0:2]
        dstb = rest[2:4]
        rows = rest[4:4 + RING]
        acc_sh = rest[4 + RING]
        isem = rest[5 + RING:7 + RING]
        gsem = rest[7 + RING:]
        c = lax.axis_index("c")
        s = lax.axis_index("s")

        pltpu.sync_copy(z_hbm, acc_sh.at[pl.ds(s * RPT, RPT), :])
        plsc.subcore_barrier()

        def run(npair, row0):
            def idx_start(b, par):
                pltpu.async_copy(
                    es_hbm.at[pl.ds(row0 + b * BLKC, BLKC), :],
                    srcb[par], isem[par])
                pltpu.async_copy(
                    ed_hbm.at[pl.ds(row0 + b * BLKC, BLKC), :],
                    dstb[par], isem[par])

            def idx_wait(b, par):
                pltpu.make_async_copy(
                    es_hbm.at[pl.ds(row0 + b * BLKC, BLKC), :],
                    srcb[par], isem[par]).wait()
                pltpu.make_async_copy(
                    ed_hbm.at[pl.ds(row0 + b * BLKC, BLKC), :],
                    dstb[par], isem[par]).wait()

            def g_start(par, k, rp):
                pltpu.async_copy(x_hbm.at[srcb[par].at[k]], rows[rp],
                                 gsem[rp])

            def g_wait(par, k, rp):
                pltpu.make_async_copy(x_hbm.at[srcb[par].at[k]], rows[rp],
                                      gsem[rp]).wait()

            idx_start(0, 0)
            idx_start(1, 1)
            idx_wait(0, 0)
            for k in range(LA):
                g_start(0, k, k & (RING - 1))

            def pair(p, _):
                for par in (0, 1):
                    b = 2 * p + par
                    for k in range(BLKC):
                        rp = k & (RING - 1)
                        g_wait(par, k, rp)
                        t = k + LA
                        tp = t & (RING - 1)
                        if t < BLKC:
                            g_start(par, t, tp)
                        elif par == 0:
                            if t == BLKC:
                                idx_wait(b + 1, 1)
                            g_start(1, t - BLKC, tp)
                        else:
                            @pl.when(p < npair - 1)
                            def _():
                                if t == BLKC:
                                    idx_wait(b + 1, 0)
                                g_start(0, t - BLKC, tp)
                        pltpu.sync_copy(rows[rp],
                                        acc_sh.at[dstb[par].at[k]],
                                        add=True)
                    @pl.when(p < npair - 1)
                    def _():
                        idx_start(b + 2, par)
                return 0
            lax.fori_loop(0, npair, pair, 0)

        @pl.when(c == 0)
        def _():
            run(NPAIR0, s * (R0 // NS))

        @pl.when(c == 1)
        def _():
            run(NPAIR1, R0 + s * ((R - R0) // NS))

        plsc.subcore_barrier()
        pltpu.sync_copy(acc_sh.at[pl.ds(s * RPT, RPT), :],
                        acc_out.at[c, pl.ds(s * RPT, RPT), :])

    return pl.kernel(body, out_type=out_type, mesh=_mesh(),
                     scratch_types=tuple(scratch))


def _sc_degree(NP, EP):
    R = EP // CH
    RW = R // NW
    NBLK = RW // BLKC
    RPT = NP // NS

    out_type = jax.ShapeDtypeStruct((NC, NP, DGW), jnp.float32)
    scratch = [
        pltpu.VMEM((BLKC, CH), jnp.int32),
        pltpu.VMEM((BLKC, CH), jnp.int32),
        pltpu.VMEM((CH, DGW), jnp.float32),
        pltpu.VMEM_SHARED((NP, DGW), jnp.float32),
        pltpu.SemaphoreType.DMA,
        pltpu.SemaphoreType.DMA,
    ]

    def body(ed_hbm, zd_hbm, ones_hbm, deg_out,
             db0, db1, ones_v, deg_sh, isem0, isem1):
        c = lax.axis_index("c")
        s = lax.axis_index("s")
        wid = c * NS + s
        dstb = (db0, db1)
        isem = (isem0, isem1)

        pltpu.sync_copy(ones_hbm, ones_v)
        pltpu.sync_copy(zd_hbm, deg_sh.at[pl.ds(s * RPT, RPT), :])

        row0 = wid * RW

        def idx_start(b, par):
            pltpu.async_copy(ed_hbm.at[pl.ds(row0 + b * BLKC, BLKC), :],
                             dstb[par], isem[par])

        def idx_wait(b, par):
            pltpu.make_async_copy(ed_hbm.at[pl.ds(row0 + b * BLKC, BLKC), :],
                                  dstb[par], isem[par]).wait()

        idx_start(0, 0)
        idx_start(1, 1)
        plsc.subcore_barrier()

        def pair(p, _):
            for par in (0, 1):
                b = 2 * p + par
                idx_wait(b, par)
                for k in range(BLKC):
                    pltpu.sync_copy(ones_v, deg_sh.at[dstb[par].at[k]],
                                    add=True)

                @pl.when(p < NBLK // 2 - 1)
                def _():
                    idx_start(b + 2, par)
            return 0
        lax.fori_loop(0, NBLK // 2, pair, 0)

        plsc.subcore_barrier()
        pltpu.sync_copy(deg_sh.at[pl.ds(s * RPT, RPT), :],
                        deg_out.at[c, pl.ds(s * RPT, RPT), :])

    return pl.kernel(body, out_type=out_type, mesh=_mesh(),
                     scratch_types=tuple(scratch))


def _tc_layer(NP, D, BLK):
    NB = NP // BLK

    def body(acc_ref, deg_ref, x_ref, w_ref, b_ref, o_ref):
        a = acc_ref[0] + acc_ref[1]
        d = deg_ref[0, :, 0] + deg_ref[1, :, 0]
        d = jnp.maximum(d, 1.0)
        z = a / d[:, None] + x_ref[...]
        z = jnp.dot(z, w_ref[...], preferred_element_type=jnp.float32)
        o_ref[...] = jnp.maximum(z + b_ref[...], 0.0)

    return pl.pallas_call(
        body,
        grid=(NB,),
        in_specs=[
            pl.BlockSpec((NC, BLK, D), lambda i: (0, i, 0)),
            pl.BlockSpec((NC, BLK, DGW), lambda i: (0, i, 0)),
            pl.BlockSpec((BLK, D), lambda i: (i, 0)),
            pl.BlockSpec((D, D), lambda i: (0, 0)),
            pl.BlockSpec((1, D), lambda i: (0, 0)),
        ],
        out_specs=pl.BlockSpec((BLK, D), lambda i: (i, 0)),
        out_shape=jax.ShapeDtypeStruct((NP, D), jnp.float32),
    )


def _sc_gather_focal(NP, D, F):
    RW = F // NS
    out_type = (jax.ShapeDtypeStruct((F, D), jnp.float32),
                jax.ShapeDtypeStruct((F, D), jnp.float32))
    scratch = (
        pltpu.VMEM((RW,), jnp.int32),
        pltpu.VMEM((RW, D), jnp.float32),
        pltpu.VMEM((RW, D), jnp.float32),
        pltpu.SemaphoreType.DMA,
    )

    def body(am_hbm, pos_hbm, focal_hbm, ff_out, sp_out,
             idx_v, rbuf, pbuf, sem):
        c = lax.axis_index("c")
        s = lax.axis_index("s")
        wid = c * NS + s

        @pl.when(wid < NS)
        def _():
            pltpu.sync_copy(focal_hbm.at[pl.ds(wid * RW, RW)], idx_v)
            pltpu.async_copy(am_hbm.at[idx_v], rbuf, sem).wait()
            pltpu.sync_copy(rbuf, ff_out.at[pl.ds(wid * RW, RW), :])

        @pl.when(wid >= NS)
        def _():
            w = wid - NS
            pltpu.sync_copy(focal_hbm.at[pl.ds(w * RW, RW)], idx_v)
            pltpu.async_copy(pos_hbm.at[idx_v], pbuf, sem).wait()
            pltpu.sync_copy(pbuf, sp_out.at[pl.ds(w * RW, RW), :])

    return pl.kernel(body, out_type=out_type, mesh=_mesh(),
                     scratch_types=scratch)


def _tc_decoder(F):
    def body(ff, st, w1, b1, w2, b2, m, o_ref):
        h1 = jnp.dot(ff[...], w1[...], preferred_element_type=jnp.float32)
        h1 = jnp.maximum(h1 + b1[...], 0.0)
        dp = jnp.dot(h1, w2[...], preferred_element_type=jnp.float32) + b2[...]
        o_ref[...] = st[...] + jnp.dot(dp, m[...],
                                       preferred_element_type=jnp.float32)

    return pl.pallas_call(
        body, out_shape=jax.ShapeDtypeStruct((F, 64), jnp.float32))


def kernel(agent_feats, edge_index, agent_pos_T, focal_indices,
           W1, b1, W2, b2, Wd1, bd1, Wd2, bd2):
    N, D = agent_feats.shape
    E = edge_index.shape[1]
    F = focal_indices.shape[0]
    T = Wd2.shape[1] // 2

    NP = -(-N // (NS * CH)) * NS * CH
    EGRAN = NW * CH * BLKC * 2
    EP = -(-E // EGRAN) * EGRAN

    x = jnp.pad(agent_feats, ((0, NP - N), (0, 0)))
    fill = jnp.full((2, EP - E), NP - 1, jnp.int32)
    ei = jnp.concatenate([edge_index.astype(jnp.int32), fill], axis=1)
    es = ei[0].reshape(EP // CH, CH)
    ed = ei[1].reshape(EP // CH, CH)
    pos = jnp.tile(jnp.pad(agent_pos_T, ((0, NP - N), (0, 0))), (1, D // 2))

    RPT = NP // NS
    zz = jnp.zeros((RPT, D), jnp.float32)
    zd = jnp.zeros((RPT, DGW), jnp.float32)
    on1 = jnp.ones((CH, DGW), jnp.float32)

    deg = _sc_degree(NP, EP)(ed, zd, on1)
    agg = _sc_aggregate(NP, D, EP, PAIR0_FRAC)
    acc1 = agg(x, es, ed, zz)
    layer = _tc_layer(NP, D, 1024)
    h = layer(acc1, deg, x, W1, b1.reshape(1, D))
    acc2 = agg(h, es, ed, zz)
    amap = layer(acc2, deg, h, W2, b2.reshape(1, D))

    ff, sp = _sc_gather_focal(NP, D, F)(amap, pos, focal_indices)

    st64 = sp[:, :64]
    w2p = jnp.pad(Wd2, ((0, 0), (0, 4)))
    b2p = jnp.pad(bd2, (0, 4)).reshape(1, 64)
    out = _tc_decoder(F)(ff, st64, Wd1, bd1.reshape(1, Wd1.shape[1]),
                         w2p, b2p, jnp.asarray(_M64))
    pred_traj = out[:, :2 * T].reshape(F, T, 2)
    return (pred_traj, focal_indices)


# speedup vs baseline: 3.3375x; 1.0680x over previous
import numpy as np
import jax
import jax.numpy as jnp
from jax import lax
from jax.experimental import pallas as pl
from jax.experimental.pallas import tpu as pltpu, tpu_sc as plsc

NC, NS, LANES = 2, 16, 16
NW = NC * NS
DGW = 128
CH = 32
BLKC = 16
RING = 8
PAIR0_FRAC = 0.7

_TT = 30
_M64 = np.zeros((64, 64), np.float32)
for _t in range(_TT):
    for _s in range(_t, _TT):
        _M64[2 * _t, 2 * _s] = 1.0
        _M64[2 * _t + 1, 2 * _s + 1] = 1.0


def _mesh():
    return plsc.VectorSubcoreMesh(core_axis_name="c", subcore_axis_name="s",
                                  num_cores=NC, num_subcores=NS)


def _sc_aggregate(NP, D, EP, pair0_frac):
    R = EP // CH
    GRAIN = NS * BLKC * 2
    UNITS = R // GRAIN
    NPAIR0 = max(1, min(UNITS - 1, round(UNITS * pair0_frac)))
    NPAIR1 = UNITS - NPAIR0
    R0 = NPAIR0 * GRAIN
    RPT = NP // NS

    LA = RING - 1
    out_type = jax.ShapeDtypeStruct((NC, NP, D), jnp.float32)
    scratch = (
        [pltpu.VMEM((BLKC, CH), jnp.int32) for _ in range(2)]
        + [pltpu.VMEM((BLKC, CH), jnp.int32) for _ in range(2)]
        + [pltpu.VMEM((CH, D), jnp.float32) for _ in range(RING)]
        + [pltpu.VMEM_SHARED((NP, D), jnp.float32)]
        + [pltpu.SemaphoreType.DMA for _ in range(2 + RING)]
    )

    def body(x_hbm, es_hbm, ed_hbm, z_hbm, acc_out, *rest):
        srcb = rest[0:2]
        dstb = rest[2:4]
        rows = rest[4:4 + RING]
        acc_sh = rest[4 + RING]
        isem = rest[5 + RING:7 + RING]
        gsem = rest[7 + RING:]
        c = lax.axis_index("c")
        s = lax.axis_index("s")

        pltpu.sync_copy(z_hbm, acc_sh.at[pl.ds(s * RPT, RPT), :])
        plsc.subcore_barrier()

        def run(npair, row0):
            def idx_start(b, par):
                pltpu.async_copy(
                    es_hbm.at[pl.ds(row0 + b * BLKC, BLKC), :],
                    srcb[par], isem[par])
                pltpu.async_copy(
                    ed_hbm.at[pl.ds(row0 + b * BLKC, BLKC), :],
                    dstb[par], isem[par])

            def idx_wait(b, par):
                pltpu.make_async_copy(
                    es_hbm.at[pl.ds(row0 + b * BLKC, BLKC), :],
                    srcb[par], isem[par]).wait()
                pltpu.make_async_copy(
                    ed_hbm.at[pl.ds(row0 + b * BLKC, BLKC), :],
                    dstb[par], isem[par]).wait()

            def g_start(par, k, rp):
                pltpu.async_copy(x_hbm.at[srcb[par].at[k]], rows[rp],
                                 gsem[rp])

            def g_wait(par, k, rp):
                pltpu.make_async_copy(x_hbm.at[srcb[par].at[k]], rows[rp],
                                      gsem[rp]).wait()

            idx_start(0, 0)
            idx_start(1, 1)
            idx_wait(0, 0)
            for k in range(LA):
                g_start(0, k, k & (RING - 1))

            def pair(p, _):
                for par in (0, 1):
                    b = 2 * p + par
                    for k in range(BLKC):
                        rp = k & (RING - 1)
                        g_wait(par, k, rp)
                        t = k + LA
                        tp = t & (RING - 1)
                        if t < BLKC:
                            g_start(par, t, tp)
                        elif par == 0:
                            if t == BLKC:
                                idx_wait(b + 1, 1)
                            g_start(1, t - BLKC, tp)
                        else:
                            @pl.when(p < npair - 1)
                            def _():
                                if t == BLKC:
                                    idx_wait(b + 1, 0)
                                g_start(0, t - BLKC, tp)
                        pltpu.sync_copy(rows[rp],
                                        acc_sh.at[dstb[par].at[k]],
                                        add=True)
                    @pl.when(p < npair - 1)
                    def _():
                        idx_start(b + 2, par)
                return 0
            lax.fori_loop(0, npair, pair, 0)

        @pl.when(c == 0)
        def _():
            run(NPAIR0, s * (R0 // NS))

        @pl.when(c == 1)
        def _():
            run(NPAIR1, R0 + s * ((R - R0) // NS))

        plsc.subcore_barrier()
        pltpu.sync_copy(acc_sh.at[pl.ds(s * RPT, RPT), :],
                        acc_out.at[c, pl.ds(s * RPT, RPT), :])

    return pl.kernel(body, out_type=out_type, mesh=_mesh(),
                     scratch_types=tuple(scratch))


def _sc_degree(NP, EP):
    R = EP // CH
    RW = R // NW
    NBLK = RW // BLKC
    RPT = NP // NS

    out_type = jax.ShapeDtypeStruct((NC, NP, DGW), jnp.float32)
    scratch = [
        pltpu.VMEM((BLKC, CH), jnp.int32),
        pltpu.VMEM((BLKC, CH), jnp.int32),
        pltpu.VMEM((CH, DGW), jnp.float32),
        pltpu.VMEM_SHARED((NP, DGW), jnp.float32),
        pltpu.SemaphoreType.DMA,
        pltpu.SemaphoreType.DMA,
    ]

    def body(ed_hbm, zd_hbm, ones_hbm, deg_out,
             db0, db1, ones_v, deg_sh, isem0, isem1):
        c = lax.axis_index("c")
        s = lax.axis_index("s")
        wid = c * NS + s
        dstb = (db0, db1)
        isem = (isem0, isem1)

        pltpu.sync_copy(ones_hbm, ones_v)
        pltpu.sync_copy(zd_hbm, deg_sh.at[pl.ds(s * RPT, RPT), :])

        row0 = wid * RW

        def idx_start(b, par):
            pltpu.async_copy(ed_hbm.at[pl.ds(row0 + b * BLKC, BLKC), :],
                             dstb[par], isem[par])

        def idx_wait(b, par):
            pltpu.make_async_copy(ed_hbm.at[pl.ds(row0 + b * BLKC, BLKC), :],
                                  dstb[par], isem[par]).wait()

        idx_start(0, 0)
        idx_start(1, 1)
        plsc.subcore_barrier()

        def pair(p, _):
            for par in (0, 1):
                b = 2 * p + par
                idx_wait(b, par)
                for k in range(BLKC):
                    pltpu.sync_copy(ones_v, deg_sh.at[dstb[par].at[k]],
                                    add=True)

                @pl.when(p < NBLK // 2 - 1)
                def _():
                    idx_start(b + 2, par)
            return 0
        lax.fori_loop(0, NBLK // 2, pair, 0)

        plsc.subcore_barrier()
        pltpu.sync_copy(deg_sh.at[pl.ds(s * RPT, RPT), :],
                        deg_out.at[c, pl.ds(s * RPT, RPT), :])

    return pl.kernel(body, out_type=out_type, mesh=_mesh(),
                     scratch_types=tuple(scratch))


def _tc_layer(NP, D, BLK):
    NB = NP // BLK

    def body(acc_ref, deg_ref, x_ref, w_ref, b_ref, o_ref):
        a = acc_ref[0] + acc_ref[1]
        d = deg_ref[0, :, 0] + deg_ref[1, :, 0]
        d = jnp.maximum(d, 1.0)
        z = a / d[:, None] + x_ref[...]
        z = jnp.dot(z, w_ref[...], preferred_element_type=jnp.float32)
        o_ref[...] = jnp.maximum(z + b_ref[...], 0.0)

    return pl.pallas_call(
        body,
        grid=(NB,),
        in_specs=[
            pl.BlockSpec((NC, BLK, D), lambda i: (0, i, 0)),
            pl.BlockSpec((NC, BLK, DGW), lambda i: (0, i, 0)),
            pl.BlockSpec((BLK, D), lambda i: (i, 0)),
            pl.BlockSpec((D, D), lambda i: (0, 0)),
            pl.BlockSpec((1, D), lambda i: (0, 0)),
        ],
        out_specs=pl.BlockSpec((BLK, D), lambda i: (i, 0)),
        out_shape=jax.ShapeDtypeStruct((NP, D), jnp.float32),
    )


def _sc_gather_focal(NP, D, F):
    RW = F // NS
    out_type = (jax.ShapeDtypeStruct((F, D), jnp.float32),
                jax.ShapeDtypeStruct((F, D), jnp.float32))
    scratch = (
        pltpu.VMEM((RW,), jnp.int32),
        pltpu.VMEM((RW, D), jnp.float32),
        pltpu.VMEM((RW, D), jnp.float32),
        pltpu.SemaphoreType.DMA,
    )

    def body(am_hbm, pos_hbm, focal_hbm, ff_out, sp_out,
             idx_v, rbuf, pbuf, sem):
        c = lax.axis_index("c")
        s = lax.axis_index("s")
        wid = c * NS + s

        @pl.when(wid < NS)
        def _():
            pltpu.sync_copy(focal_hbm.at[pl.ds(wid * RW, RW)], idx_v)
            pltpu.async_copy(am_hbm.at[idx_v], rbuf, sem).wait()
            pltpu.sync_copy(rbuf, ff_out.at[pl.ds(wid * RW, RW), :])

        @pl.when(wid >= NS)
        def _():
            w = wid - NS
            pltpu.sync_copy(focal_hbm.at[pl.ds(w * RW, RW)], idx_v)
            pltpu.async_copy(pos_hbm.at[idx_v], pbuf, sem).wait()
            pltpu.sync_copy(pbuf, sp_out.at[pl.ds(w * RW, RW), :])

    return pl.kernel(body, out_type=out_type, mesh=_mesh(),
                     scratch_types=scratch)


def _tc_decoder(F):
    def body(ff, st, w1, b1, w2, b2, m, o_ref):
        h1 = jnp.dot(ff[...], w1[...], preferred_element_type=jnp.float32)
        h1 = jnp.maximum(h1 + b1[...], 0.0)
        dp = jnp.dot(h1, w2[...], preferred_element_type=jnp.float32) + b2[...]
        o_ref[...] = st[...] + jnp.dot(dp, m[...],
                                       preferred_element_type=jnp.float32)

    return pl.pallas_call(
        body, out_shape=jax.ShapeDtypeStruct((F, 64), jnp.float32))


def kernel(agent_feats, edge_index, agent_pos_T, focal_indices,
           W1, b1, W2, b2, Wd1, bd1, Wd2, bd2):
    N, D = agent_feats.shape
    E = edge_index.shape[1]
    F = focal_indices.shape[0]
    T = Wd2.shape[1] // 2

    NP = -(-N // (NS * CH)) * NS * CH
    EGRAN = NW * CH * BLKC * 2
    EP = -(-E // EGRAN) * EGRAN

    x = jnp.pad(agent_feats, ((0, NP - N), (0, 0)))
    fill = jnp.full((2, EP - E), NP - 1, jnp.int32)
    ei = jnp.concatenate([edge_index.astype(jnp.int32), fill], axis=1)
    es = ei[0].reshape(EP // CH, CH)
    ed = ei[1].reshape(EP // CH, CH)
    pos = jnp.tile(jnp.pad(agent_pos_T, ((0, NP - N), (0, 0))), (1, D // 2))

    RPT = NP // NS
    zz = jnp.zeros((RPT, D), jnp.float32)
    zd = jnp.zeros((RPT, DGW), jnp.float32)
    on1 = jnp.ones((CH, DGW), jnp.float32)

    deg = _sc_degree(NP, EP)(ed, zd, on1)
    agg = _sc_aggregate(NP, D, EP, PAIR0_FRAC)
    acc1 = agg(x, es, ed, zz)
    layer = _tc_layer(NP, D, 1024)
    h = layer(acc1, deg, x, W1, b1.reshape(1, D))
    acc2 = agg(h, es, ed, zz)
    amap = layer(acc2, deg, h, W2, b2.reshape(1, D))

    ff, sp = _sc_gather_focal(NP, D, F)(amap, pos, focal_indices)

    st64 = sp[:, :64]
    w2p = jnp.pad(Wd2, ((0, 0), (0, 4)))
    b2p = jnp.pad(bd2, (0, 4)).reshape(1, 64)
    out = _tc_decoder(F)(ff, st64, Wd1, bd1.reshape(1, Wd1.shape[1]),
                         w2p, b2p, jnp.asarray(_M64))
    pred_traj = out[:, :2 * T].reshape(F, T, 2)
    return (pred_traj, focal_indices)


# speedup vs baseline: 4.7298x; 1.4171x over previous
import numpy as np
import jax
import jax.numpy as jnp
from jax import lax
from jax.experimental import pallas as pl
from jax.experimental.pallas import tpu as pltpu, tpu_sc as plsc

NC, NS, LANES = 2, 16, 16
NW = NC * NS
DGW = 128
CH = 32
BLKC = 16
RING = 8
PAIR0_FRAC = 0.7

_TT = 30
_M64 = np.zeros((64, 64), np.float32)
for _t in range(_TT):
    for _s in range(_t, _TT):
        _M64[2 * _t, 2 * _s] = 1.0
        _M64[2 * _t + 1, 2 * _s + 1] = 1.0

_G128 = np.zeros((128, 128), np.float32)
for _l in range(128):
    _G128[_l, _l // CH] = 1.0


def _mesh():
    return plsc.VectorSubcoreMesh(core_axis_name="c", subcore_axis_name="s",
                                  num_cores=NC, num_subcores=NS)


def _tc_chunk_flags(R2, F):
    def body(ed_ref, focal_ref, g_ref, o_ref):
        ed = ed_ref[...]
        m = jnp.zeros(ed.shape, jnp.float32)
        for j in range(F):
            m = m + (ed == focal_ref[0, j]).astype(jnp.float32)
        o_ref[...] = jnp.dot(m, g_ref[...], preferred_element_type=jnp.float32)

    return pl.pallas_call(
        body, out_shape=jax.ShapeDtypeStruct((R2, 128), jnp.float32))


def _sc_aggregate(NP, D, EP, pair0_frac, filtered=False):
    R = EP // CH
    GRAIN = NS * BLKC * 2
    UNITS = R // GRAIN
    NPAIR0 = max(1, min(UNITS - 1, round(UNITS * pair0_frac)))
    NPAIR1 = UNITS - NPAIR0
    R0 = NPAIR0 * GRAIN
    RPT = NP // NS

    LA = RING - 1
    out_type = jax.ShapeDtypeStruct((NC, NP, D), jnp.float32)
    scratch = (
        [pltpu.VMEM((BLKC, CH), jnp.int32) for _ in range(2)]
        + [pltpu.VMEM((BLKC, CH), jnp.int32) for _ in range(2)]
        + [pltpu.VMEM((BLKC,), jnp.float32) for _ in range(2)]
        + [pltpu.VMEM((CH, D), jnp.float32) for _ in range(RING)]
        + [pltpu.VMEM_SHARED((NP, D), jnp.float32)]
        + [pltpu.SemaphoreType.DMA for _ in range(2 + RING)]
    )

    def body(x_hbm, es_hbm, ed_hbm, cf_hbm, z_hbm, acc_out, *rest):
        srcb = rest[0:2]
        dstb = rest[2:4]
        fb = rest[4:6]
        rows = rest[6:6 + RING]
        acc_sh = rest[6 + RING]
        isem = rest[7 + RING:9 + RING]
        gsem = rest[9 + RING:]
        c = lax.axis_index("c")
        s = lax.axis_index("s")

        pltpu.sync_copy(z_hbm, acc_sh.at[pl.ds(s * RPT, RPT), :])
        plsc.subcore_barrier()

        def run(npair, row0):
            def idx_start(b, par):
                pltpu.async_copy(
                    es_hbm.at[pl.ds(row0 + b * BLKC, BLKC), :],
                    srcb[par], isem[par])
                pltpu.async_copy(
                    ed_hbm.at[pl.ds(row0 + b * BLKC, BLKC), :],
                    dstb[par], isem[par])
                if filtered:
                    pltpu.async_copy(
                        cf_hbm.at[pl.ds(row0 + b * BLKC, BLKC)],
                        fb[par], isem[par])

            def idx_wait(b, par):
                pltpu.make_async_copy(
                    es_hbm.at[pl.ds(row0 + b * BLKC, BLKC), :],
                    srcb[par], isem[par]).wait()
                pltpu.make_async_copy(
                    ed_hbm.at[pl.ds(row0 + b * BLKC, BLKC), :],
                    dstb[par], isem[par]).wait()
                if filtered:
                    pltpu.make_async_copy(
                        cf_hbm.at[pl.ds(row0 + b * BLKC, BLKC)],
                        fb[par], isem[par]).wait()

            def live(par, k):
                lv = fb[par][pl.ds(0, BLKC)]
                return lv[k] > 0.5

            def g_start(par, k, rp):
                def do():
                    pltpu.async_copy(x_hbm.at[srcb[par].at[k]], rows[rp],
                                     gsem[rp])
                if filtered:
                    pl.when(live(par, k))(do)
                else:
                    do()

            def g_wait(par, k, rp):
                def do():
                    pltpu.make_async_copy(x_hbm.at[srcb[par].at[k]],
                                          rows[rp], gsem[rp]).wait()
                if filtered:
                    pl.when(live(par, k))(do)
                else:
                    do()

            idx_start(0, 0)
            idx_start(1, 1)
            idx_wait(0, 0)
            for k in range(LA):
                g_start(0, k, k & (RING - 1))

            def pair(p, _):
                for par in (0, 1):
                    b = 2 * p + par
                    for k in range(BLKC):
                        rp = k & (RING - 1)
                        g_wait(par, k, rp)
                        t = k + LA
                        tp = t & (RING - 1)
                        if t < BLKC:
                            g_start(par, t, tp)
                        elif par == 0:
                            if t == BLKC:
                                idx_wait(b + 1, 1)
                            g_start(1, t - BLKC, tp)
                        else:
                            @pl.when(p < npair - 1)
                            def _():
                                if t == BLKC:
                                    idx_wait(b + 1, 0)
                                g_start(0, t - BLKC, tp)
                        def scat(par=par, k=k, rp=rp):
                            pltpu.sync_copy(rows[rp],
                                            acc_sh.at[dstb[par].at[k]],
                                            add=True)
                        if filtered:
                            pl.when(live(par, k))(scat)
                        else:
                            scat()
                    @pl.when(p < npair - 1)
                    def _():
                        idx_start(b + 2, par)
                return 0
            lax.fori_loop(0, npair, pair, 0)

        @pl.when(c == 0)
        def _():
            run(NPAIR0, s * (R0 // NS))

        @pl.when(c == 1)
        def _():
            run(NPAIR1, R0 + s * ((R - R0) // NS))

        plsc.subcore_barrier()
        pltpu.sync_copy(acc_sh.at[pl.ds(s * RPT, RPT), :],
                        acc_out.at[c, pl.ds(s * RPT, RPT), :])

    return pl.kernel(body, out_type=out_type, mesh=_mesh(),
                     scratch_types=tuple(scratch))


def _sc_degree(NP, EP):
    R = EP // CH
    RW = R // NW
    NBLK = RW // BLKC
    RPT = NP // NS

    out_type = jax.ShapeDtypeStruct((NC, NP, DGW), jnp.float32)
    scratch = [
        pltpu.VMEM((BLKC, CH), jnp.int32),
        pltpu.VMEM((BLKC, CH), jnp.int32),
        pltpu.VMEM((CH, DGW), jnp.float32),
        pltpu.VMEM_SHARED((NP, DGW), jnp.float32),
        pltpu.SemaphoreType.DMA,
        pltpu.SemaphoreType.DMA,
    ]

    def body(ed_hbm, zd_hbm, ones_hbm, deg_out,
             db0, db1, ones_v, deg_sh, isem0, isem1):
        c = lax.axis_index("c")
        s = lax.axis_index("s")
        wid = c * NS + s
        dstb = (db0, db1)
        isem = (isem0, isem1)

        pltpu.sync_copy(ones_hbm, ones_v)
        pltpu.sync_copy(zd_hbm, deg_sh.at[pl.ds(s * RPT, RPT), :])

        row0 = wid * RW

        def idx_start(b, par):
            pltpu.async_copy(ed_hbm.at[pl.ds(row0 + b * BLKC, BLKC), :],
                             dstb[par], isem[par])

        def idx_wait(b, par):
            pltpu.make_async_copy(ed_hbm.at[pl.ds(row0 + b * BLKC, BLKC), :],
                                  dstb[par], isem[par]).wait()

        idx_start(0, 0)
        idx_start(1, 1)
        plsc.subcore_barrier()

        def pair(p, _):
            for par in (0, 1):
                b = 2 * p + par
                idx_wait(b, par)
                for k in range(BLKC):
                    pltpu.sync_copy(ones_v, deg_sh.at[dstb[par].at[k]],
                                    add=True)

                @pl.when(p < NBLK // 2 - 1)
                def _():
                    idx_start(b + 2, par)
            return 0
        lax.fori_loop(0, NBLK // 2, pair, 0)

        plsc.subcore_barrier()
        pltpu.sync_copy(deg_sh.at[pl.ds(s * RPT, RPT), :],
                        deg_out.at[c, pl.ds(s * RPT, RPT), :])

    return pl.kernel(body, out_type=out_type, mesh=_mesh(),
                     scratch_types=tuple(scratch))


def _tc_layer(NP, D, BLK):
    NB = NP // BLK

    def body(acc_ref, deg_ref, x_ref, w_ref, b_ref, o_ref):
        a = acc_ref[0] + acc_ref[1]
        d = deg_ref[0, :, 0] + deg_ref[1, :, 0]
        d = jnp.maximum(d, 1.0)
        z = a / d[:, None] + x_ref[...]
        z = jnp.dot(z, w_ref[...], preferred_element_type=jnp.float32)
        o_ref[...] = jnp.maximum(z + b_ref[...], 0.0)

    return pl.pallas_call(
        body,
        grid=(NB,),
        in_specs=[
            pl.BlockSpec((NC, BLK, D), lambda i: (0, i, 0)),
            pl.BlockSpec((NC, BLK, DGW), lambda i: (0, i, 0)),
            pl.BlockSpec((BLK, D), lambda i: (i, 0)),
            pl.BlockSpec((D, D), lambda i: (0, 0)),
            pl.BlockSpec((1, D), lambda i: (0, 0)),
        ],
        out_specs=pl.BlockSpec((BLK, D), lambda i: (i, 0)),
        out_shape=jax.ShapeDtypeStruct((NP, D), jnp.float32),
    )


def _sc_gather_focal(NP, D, F):
    RW = F // NS
    out_type = (jax.ShapeDtypeStruct((F, D), jnp.float32),
                jax.ShapeDtypeStruct((F, D), jnp.float32))
    scratch = (
        pltpu.VMEM((RW,), jnp.int32),
        pltpu.VMEM((RW, D), jnp.float32),
        pltpu.VMEM((RW, D), jnp.float32),
        pltpu.SemaphoreType.DMA,
    )

    def body(am_hbm, pos_hbm, focal_hbm, ff_out, sp_out,
             idx_v, rbuf, pbuf, sem):
        c = lax.axis_index("c")
        s = lax.axis_index("s")
        wid = c * NS + s

        @pl.when(wid < NS)
        def _():
            pltpu.sync_copy(focal_hbm.at[pl.ds(wid * RW, RW)], idx_v)
            pltpu.async_copy(am_hbm.at[idx_v], rbuf, sem).wait()
            pltpu.sync_copy(rbuf, ff_out.at[pl.ds(wid * RW, RW), :])

        @pl.when(wid >= NS)
        def _():
            w = wid - NS
            pltpu.sync_copy(focal_hbm.at[pl.ds(w * RW, RW)], idx_v)
            pltpu.async_copy(pos_hbm.at[idx_v], pbuf, sem).wait()
            pltpu.sync_copy(pbuf, sp_out.at[pl.ds(w * RW, RW), :])

    return pl.kernel(body, out_type=out_type, mesh=_mesh(),
                     scratch_types=scratch)


def _tc_decoder(F):
    def body(ff, st, w1, b1, w2, b2, m, o_ref):
        h1 = jnp.dot(ff[...], w1[...], preferred_element_type=jnp.float32)
        h1 = jnp.maximum(h1 + b1[...], 0.0)
        dp = jnp.dot(h1, w2[...], preferred_element_type=jnp.float32) + b2[...]
        o_ref[...] = st[...] + jnp.dot(dp, m[...],
                                       preferred_element_type=jnp.float32)

    return pl.pallas_call(
        body, out_shape=jax.ShapeDtypeStruct((F, 64), jnp.float32))


def kernel(agent_feats, edge_index, agent_pos_T, focal_indices,
           W1, b1, W2, b2, Wd1, bd1, Wd2, bd2):
    N, D = agent_feats.shape
    E = edge_index.shape[1]
    F = focal_indices.shape[0]
    T = Wd2.shape[1] // 2

    NP = -(-N // (NS * CH)) * NS * CH
    EGRAN = NW * CH * BLKC * 2
    EP = -(-E // EGRAN) * EGRAN

    x = jnp.pad(agent_feats, ((0, NP - N), (0, 0)))
    fill = jnp.full((2, EP - E), NP - 1, jnp.int32)
    ei = jnp.concatenate([edge_index.astype(jnp.int32), fill], axis=1)
    es = ei[0].reshape(EP // CH, CH)
    ed = ei[1].reshape(EP // CH, CH)
    pos = jnp.tile(jnp.pad(agent_pos_T, ((0, NP - N), (0, 0))), (1, D // 2))

    RPT = NP // NS
    zz = jnp.zeros((RPT, D), jnp.float32)
    zd = jnp.zeros((RPT, DGW), jnp.float32)
    on1 = jnp.ones((CH, DGW), jnp.float32)

    R = EP // CH
    fl = _tc_chunk_flags(EP // 128, F)(ed.reshape(EP // 128, 128),
                                       focal_indices.reshape(1, F),
                                       jnp.asarray(_G128))
    cf = fl[:, :128 // CH].reshape(R)

    deg = _sc_degree(NP, EP)(ed, zd, on1)
    acc1 = _sc_aggregate(NP, D, EP, PAIR0_FRAC)(x, es, ed, cf, zz)
    layer = _tc_layer(NP, D, 1024)
    h = layer(acc1, deg, x, W1, b1.reshape(1, D))
    acc2 = _sc_aggregate(NP, D, EP, PAIR0_FRAC, True)(h, es, ed, cf, zz)
    amap = layer(acc2, deg, h, W2, b2.reshape(1, D))

    ff, sp = _sc_gather_focal(NP, D, F)(amap, pos, focal_indices)

    st64 = sp[:, :64]
    w2p = jnp.pad(Wd2, ((0, 0), (0, 4)))
    b2p = jnp.pad(bd2, (0, 4)).reshape(1, 64)
    out = _tc_decoder(F)(ff, st64, Wd1, bd1.reshape(1, Wd1.shape[1]),
                         w2p, b2p, jnp.asarray(_M64))
    pred_traj = out[:, :2 * T].reshape(F, T, 2)
    return (pred_traj, focal_indices)


# speedup vs baseline: 4.7548x; 1.0053x over previous
import numpy as np
import jax
import jax.numpy as jnp
from jax import lax
from jax.experimental import pallas as pl
from jax.experimental.pallas import tpu as pltpu, tpu_sc as plsc

NC, NS, LANES = 2, 16, 16
NW = NC * NS
DGW = 128
CH = 32
BLKC = 16
RING = 8
PAIR0_FRAC = 0.6

_TT = 30
_M64 = np.zeros((64, 64), np.float32)
for _t in range(_TT):
    for _s in range(_t, _TT):
        _M64[2 * _t, 2 * _s] = 1.0
        _M64[2 * _t + 1, 2 * _s + 1] = 1.0

_G128 = np.zeros((128, 128), np.float32)
for _l in range(128):
    _G128[_l, _l // CH] = 1.0


def _mesh():
    return plsc.VectorSubcoreMesh(core_axis_name="c", subcore_axis_name="s",
                                  num_cores=NC, num_subcores=NS)


def _tc_chunk_flags(R2, F):
    def body(ed_ref, focal_ref, g_ref, o_ref):
        ed = ed_ref[...]
        m = jnp.zeros(ed.shape, jnp.float32)
        for j in range(F):
            m = m + (ed == focal_ref[0, j]).astype(jnp.float32)
        o_ref[...] = jnp.dot(m, g_ref[...], preferred_element_type=jnp.float32)

    return pl.pallas_call(
        body, out_shape=jax.ShapeDtypeStruct((R2, 128), jnp.float32))


def _sc_aggregate(NP, D, EP, pair0_frac, filtered=False):
    R = EP // CH
    GRAIN = NS * BLKC * 2
    UNITS = R // GRAIN
    NPAIR0 = max(1, min(UNITS - 1, round(UNITS * pair0_frac)))
    NPAIR1 = UNITS - NPAIR0
    R0 = NPAIR0 * GRAIN
    RPT = NP // NS

    LA = RING - 1
    out_type = jax.ShapeDtypeStruct((NC, NP, D), jnp.float32)
    scratch = (
        [pltpu.VMEM((BLKC, CH), jnp.int32) for _ in range(2)]
        + [pltpu.VMEM((BLKC, CH), jnp.int32) for _ in range(2)]
        + [pltpu.VMEM((BLKC,), jnp.float32) for _ in range(2)]
        + [pltpu.VMEM((CH, D), jnp.float32) for _ in range(RING)]
        + [pltpu.VMEM_SHARED((NP, D), jnp.float32)]
        + [pltpu.SemaphoreType.DMA for _ in range(2 + RING)]
    )

    def body(x_hbm, es_hbm, ed_hbm, cf_hbm, z_hbm, acc_out, *rest):
        srcb = rest[0:2]
        dstb = rest[2:4]
        fb = rest[4:6]
        rows = rest[6:6 + RING]
        acc_sh = rest[6 + RING]
        isem = rest[7 + RING:9 + RING]
        gsem = rest[9 + RING:]
        c = lax.axis_index("c")
        s = lax.axis_index("s")

        pltpu.sync_copy(z_hbm, acc_sh.at[pl.ds(s * RPT, RPT), :])
        plsc.subcore_barrier()

        def run(npair, row0):
            def idx_start(b, par):
                pltpu.async_copy(
                    es_hbm.at[pl.ds(row0 + b * BLKC, BLKC), :],
                    srcb[par], isem[par])
                pltpu.async_copy(
                    ed_hbm.at[pl.ds(row0 + b * BLKC, BLKC), :],
                    dstb[par], isem[par])
                if filtered:
                    pltpu.async_copy(
                        cf_hbm.at[pl.ds(row0 + b * BLKC, BLKC)],
                        fb[par], isem[par])

            def idx_wait(b, par):
                pltpu.make_async_copy(
                    es_hbm.at[pl.ds(row0 + b * BLKC, BLKC), :],
                    srcb[par], isem[par]).wait()
                pltpu.make_async_copy(
                    ed_hbm.at[pl.ds(row0 + b * BLKC, BLKC), :],
                    dstb[par], isem[par]).wait()
                if filtered:
                    pltpu.make_async_copy(
                        cf_hbm.at[pl.ds(row0 + b * BLKC, BLKC)],
                        fb[par], isem[par]).wait()

            def live(par, k):
                lv = fb[par][pl.ds(0, BLKC)]
                return lv[k] > 0.5

            def g_start(par, k, rp):
                def do():
                    pltpu.async_copy(x_hbm.at[srcb[par].at[k]], rows[rp],
                                     gsem[rp])
                if filtered:
                    pl.when(live(par, k))(do)
                else:
                    do()

            def g_wait(par, k, rp):
                def do():
                    pltpu.make_async_copy(x_hbm.at[srcb[par].at[k]],
                                          rows[rp], gsem[rp]).wait()
                if filtered:
                    pl.when(live(par, k))(do)
                else:
                    do()

            idx_start(0, 0)
            idx_start(1, 1)
            idx_wait(0, 0)
            for k in range(LA):
                g_start(0, k, k & (RING - 1))

            def pair(p, _):
                for par in (0, 1):
                    b = 2 * p + par
                    for k in range(BLKC):
                        rp = k & (RING - 1)
                        g_wait(par, k, rp)
                        t = k + LA
                        tp = t & (RING - 1)
                        if t < BLKC:
                            g_start(par, t, tp)
                        elif par == 0:
                            if t == BLKC:
                                idx_wait(b + 1, 1)
                            g_start(1, t - BLKC, tp)
                        else:
                            @pl.when(p < npair - 1)
                            def _():
                                if t == BLKC:
                                    idx_wait(b + 1, 0)
                                g_start(0, t - BLKC, tp)
                        def scat(par=par, k=k, rp=rp):
                            pltpu.sync_copy(rows[rp],
                                            acc_sh.at[dstb[par].at[k]],
                                            add=True)
                        if filtered:
                            pl.when(live(par, k))(scat)
                        else:
                            scat()
                    @pl.when(p < npair - 1)
                    def _():
                        idx_start(b + 2, par)
                return 0
            lax.fori_loop(0, npair, pair, 0)

        @pl.when(c == 0)
        def _():
            run(NPAIR0, s * (R0 // NS))

        @pl.when(c == 1)
        def _():
            run(NPAIR1, R0 + s * ((R - R0) // NS))

        plsc.subcore_barrier()
        pltpu.sync_copy(acc_sh.at[pl.ds(s * RPT, RPT), :],
                        acc_out.at[c, pl.ds(s * RPT, RPT), :])

    return pl.kernel(body, out_type=out_type, mesh=_mesh(),
                     scratch_types=tuple(scratch))


def _sc_degree(NP, EP):
    R = EP // CH
    RW = R // NW
    NBLK = RW // BLKC
    RPT = NP // NS

    out_type = jax.ShapeDtypeStruct((NC, NP, DGW), jnp.float32)
    scratch = [
        pltpu.VMEM((BLKC, CH), jnp.int32),
        pltpu.VMEM((BLKC, CH), jnp.int32),
        pltpu.VMEM((CH, DGW), jnp.float32),
        pltpu.VMEM_SHARED((NP, DGW), jnp.float32),
        pltpu.SemaphoreType.DMA,
        pltpu.SemaphoreType.DMA,
    ]

    def body(ed_hbm, zd_hbm, ones_hbm, deg_out,
             db0, db1, ones_v, deg_sh, isem0, isem1):
        c = lax.axis_index("c")
        s = lax.axis_index("s")
        wid = c * NS + s
        dstb = (db0, db1)
        isem = (isem0, isem1)

        pltpu.sync_copy(ones_hbm, ones_v)
        pltpu.sync_copy(zd_hbm, deg_sh.at[pl.ds(s * RPT, RPT), :])

        row0 = wid * RW

        def idx_start(b, par):
            pltpu.async_copy(ed_hbm.at[pl.ds(row0 + b * BLKC, BLKC), :],
                             dstb[par], isem[par])

        def idx_wait(b, par):
            pltpu.make_async_copy(ed_hbm.at[pl.ds(row0 + b * BLKC, BLKC), :],
                                  dstb[par], isem[par]).wait()

        idx_start(0, 0)
        idx_start(1, 1)
        plsc.subcore_barrier()

        def pair(p, _):
            for par in (0, 1):
                b = 2 * p + par
                idx_wait(b, par)
                for k in range(BLKC):
                    pltpu.sync_copy(ones_v, deg_sh.at[dstb[par].at[k]],
                                    add=True)

                @pl.when(p < NBLK // 2 - 1)
                def _():
                    idx_start(b + 2, par)
            return 0
        lax.fori_loop(0, NBLK // 2, pair, 0)

        plsc.subcore_barrier()
        pltpu.sync_copy(deg_sh.at[pl.ds(s * RPT, RPT), :],
                        deg_out.at[c, pl.ds(s * RPT, RPT), :])

    return pl.kernel(body, out_type=out_type, mesh=_mesh(),
                     scratch_types=tuple(scratch))


def _tc_layer(NP, D, BLK):
    NB = NP // BLK

    def body(acc_ref, deg_ref, x_ref, w_ref, b_ref, o_ref):
        a = acc_ref[0] + acc_ref[1]
        d = deg_ref[0, :, 0] + deg_ref[1, :, 0]
        d = jnp.maximum(d, 1.0)
        z = a / d[:, None] + x_ref[...]
        z = jnp.dot(z, w_ref[...], preferred_element_type=jnp.float32)
        o_ref[...] = jnp.maximum(z + b_ref[...], 0.0)

    return pl.pallas_call(
        body,
        grid=(NB,),
        in_specs=[
            pl.BlockSpec((NC, BLK, D), lambda i: (0, i, 0)),
            pl.BlockSpec((NC, BLK, DGW), lambda i: (0, i, 0)),
            pl.BlockSpec((BLK, D), lambda i: (i, 0)),
            pl.BlockSpec((D, D), lambda i: (0, 0)),
            pl.BlockSpec((1, D), lambda i: (0, 0)),
        ],
        out_specs=pl.BlockSpec((BLK, D), lambda i: (i, 0)),
        out_shape=jax.ShapeDtypeStruct((NP, D), jnp.float32),
    )


def _sc_gather_focal(NP, D, F):
    RW = F // NS
    out_type = (jax.ShapeDtypeStruct((F, D), jnp.float32),
                jax.ShapeDtypeStruct((F, D), jnp.float32))
    scratch = (
        pltpu.VMEM((RW,), jnp.int32),
        pltpu.VMEM((RW, D), jnp.float32),
        pltpu.VMEM((RW, D), jnp.float32),
        pltpu.SemaphoreType.DMA,
    )

    def body(am_hbm, pos_hbm, focal_hbm, ff_out, sp_out,
             idx_v, rbuf, pbuf, sem):
        c = lax.axis_index("c")
        s = lax.axis_index("s")
        wid = c * NS + s

        @pl.when(wid < NS)
        def _():
            pltpu.sync_copy(focal_hbm.at[pl.ds(wid * RW, RW)], idx_v)
            pltpu.async_copy(am_hbm.at[idx_v], rbuf, sem).wait()
            pltpu.sync_copy(rbuf, ff_out.at[pl.ds(wid * RW, RW), :])

        @pl.when(wid >= NS)
        def _():
            w = wid - NS
            pltpu.sync_copy(focal_hbm.at[pl.ds(w * RW, RW)], idx_v)
            pltpu.async_copy(pos_hbm.at[idx_v], pbuf, sem).wait()
            pltpu.sync_copy(pbuf, sp_out.at[pl.ds(w * RW, RW), :])

    return pl.kernel(body, out_type=out_type, mesh=_mesh(),
                     scratch_types=scratch)


def _tc_decoder(F):
    def body(ff, st, w1, b1, w2, b2, m, o_ref):
        h1 = jnp.dot(ff[...], w1[...], preferred_element_type=jnp.float32)
        h1 = jnp.maximum(h1 + b1[...], 0.0)
        dp = jnp.dot(h1, w2[...], preferred_element_type=jnp.float32) + b2[...]
        o_ref[...] = st[...] + jnp.dot(dp, m[...],
                                       preferred_element_type=jnp.float32)

    return pl.pallas_call(
        body, out_shape=jax.ShapeDtypeStruct((F, 64), jnp.float32))


def kernel(agent_feats, edge_index, agent_pos_T, focal_indices,
           W1, b1, W2, b2, Wd1, bd1, Wd2, bd2):
    N, D = agent_feats.shape
    E = edge_index.shape[1]
    F = focal_indices.shape[0]
    T = Wd2.shape[1] // 2

    NP = -(-N // (NS * CH)) * NS * CH
    EGRAN = NW * CH * BLKC * 2
    EP = -(-E // EGRAN) * EGRAN

    x = jnp.pad(agent_feats, ((0, NP - N), (0, 0)))
    fill = jnp.full((2, EP - E), NP - 1, jnp.int32)
    ei = jnp.concatenate([edge_index.astype(jnp.int32), fill], axis=1)
    es = ei[0].reshape(EP // CH, CH)
    ed = ei[1].reshape(EP // CH, CH)
    pos = jnp.tile(jnp.pad(agent_pos_T, ((0, NP - N), (0, 0))), (1, D // 2))

    RPT = NP // NS
    zz = jnp.zeros((RPT, D), jnp.float32)
    zd = jnp.zeros((RPT, DGW), jnp.float32)
    on1 = jnp.ones((CH, DGW), jnp.float32)

    R = EP // CH
    fl = _tc_chunk_flags(EP // 128, F)(ed.reshape(EP // 128, 128),
                                       focal_indices.reshape(1, F),
                                       jnp.asarray(_G128))
    cf = fl[:, :128 // CH].reshape(R)

    deg = _sc_degree(NP, EP)(ed, zd, on1)
    acc1 = _sc_aggregate(NP, D, EP, PAIR0_FRAC)(x, es, ed, cf, zz)
    layer = _tc_layer(NP, D, 1024)
    h = layer(acc1, deg, x, W1, b1.reshape(1, D))
    acc2 = _sc_aggregate(NP, D, EP, PAIR0_FRAC, True)(h, es, ed, cf, zz)
    amap = layer(acc2, deg, h, W2, b2.reshape(1, D))

    ff, sp = _sc_gather_focal(NP, D, F)(amap, pos, focal_indices)

    st64 = sp[:, :64]
    w2p = jnp.pad(Wd2, ((0, 0), (0, 4)))
    b2p = jnp.pad(bd2, (0, 4)).reshape(1, 64)
    out = _tc_decoder(F)(ff, st64, Wd1, bd1.reshape(1, Wd1.shape[1]),
                         w2p, b2p, jnp.asarray(_M64))
    pred_traj = out[:, :2 * T].reshape(F, T, 2)
    return (pred_traj, focal_indices)
